# baseline (speedup 1.0000x reference)
import functools

import jax
import jax.numpy as jnp
from jax import lax
from jax.experimental import pallas as pl
from jax.experimental.pallas import tpu as pltpu
from jax.experimental.pallas import tpu_sc as plsc

_NC = 2
_NS = 16
_NW = _NC * _NS


def _cast_bf16(W, BH):
    H, K = W.shape

    def body(w_ref, o_ref):
        o_ref[...] = w_ref[...].astype(jnp.bfloat16)

    return pl.pallas_call(
        body,
        grid=(H // BH,),
        in_specs=[pl.BlockSpec((BH, K), lambda h: (h, 0))],
        out_specs=pl.BlockSpec((BH, K), lambda h: (h, 0)),
        out_shape=jax.ShapeDtypeStruct((H, K), jnp.bfloat16),
    )(W)


def _sc_gather_tiled(table, idx3, B, S):
    V, D = table.shape
    NW, NCH, CH = idx3.shape
    assert NCH % 2 == 0 and CH % 8 == 0
    b_per_w = NCH * CH
    NT = CH // 8

    mesh = plsc.VectorSubcoreMesh(core_axis_name="c", subcore_axis_name="s")
    NSLOT = 4
    assert NCH % NSLOT == 0

    @functools.partial(
        pl.kernel,
        mesh=mesh,
        compiler_params=pltpu.CompilerParams(use_tc_tiling_on_sc=True),
        out_type=jax.ShapeDtypeStruct((B, S * D), table.dtype),
        scratch_types=[
            pltpu.VMEM((NCH, CH), jnp.int32),
            pltpu.VMEM((4, CH, D), table.dtype),
        ] + [pltpu.SemaphoreType.DMA] * 8,
    )
    def k(table_hbm, idx_hbm, out_hbm, idx_v, rows_v, *sems):
        gsems = sems[:NSLOT]
        osems = sems[NSLOT:]
        c = lax.axis_index("c")
        s = lax.axis_index("s")
        wid = s * _NC + c
        base = wid * b_per_w
        pltpu.sync_copy(idx_hbm.at[wid], idx_v)
        pltpu.async_copy(table_hbm.at[idx_v.at[0]], rows_v.at[0], gsems[0])
        pltpu.async_copy(table_hbm.at[idx_v.at[1]], rows_v.at[1], gsems[1])

        def body(g, carry):
            for b in range(NSLOT):
                j = NSLOT * g + b
                pltpu.make_async_copy(
                    table_hbm.at[pl.ds(0, CH)], rows_v.at[b], gsems[b]
                ).wait()
                t0 = (base + j * CH) // 8
                for kk in range(NT):
                    t = t0 + kk
                    bt = t // S
                    cc = t % S
                    pltpu.async_copy(
                        rows_v.at[b, pl.ds(kk * 8, 8)],
                        out_hbm.at[pl.ds(bt * 8, 8), pl.ds(cc * D, D)],
                        osems[b],
                    )
                jn = j + 2
                bn = (b + 2) % NSLOT

                @pl.when(jnp.logical_and(jn < NCH, jn >= NSLOT))
                def _():
                    pltpu.make_async_copy(
                        table_hbm.at[pl.ds(0, CH)], rows_v.at[bn], osems[bn]
                    ).wait()

                @pl.when(jn < NCH)
                def _():
                    pltpu.async_copy(
                        table_hbm.at[idx_v.at[jn]], rows_v.at[bn], gsems[bn]
                    )

            return carry

        lax.fori_loop(0, NCH // NSLOT, body, 0)

        for b in range(NSLOT):
            pltpu.make_async_copy(
                table_hbm.at[pl.ds(0, CH)], rows_v.at[b], osems[b]
            ).wait()

    return k(table, idx3)


def _tc_mlp(flat, W1b, b1r, W2b, b2r, B, block_off, out_prev=None):
    Bc, K = flat.shape
    HID = W1b.shape[0]
    NOUT = W2b.shape[0]
    BM = 256
    Mc = Bc // BM

    def body(*refs):
        x_ref, w1_ref, b1_ref, w2_ref, b2_ref = refs[:5]
        o_ref = refs[-1]
        x = x_ref[...].astype(jnp.bfloat16)
        h = lax.dot_general(
            x, w1_ref[...], (((1,), (1,)), ((), ())),
            preferred_element_type=jnp.float32,
        )
        h = jnp.maximum(h + b1_ref[...], 0.0).astype(jnp.bfloat16)
        o = lax.dot_general(
            h, w2_ref[...], (((1,), (1,)), ((), ())),
            preferred_element_type=jnp.float32,
        )
        o_ref[...] = o + b2_ref[...]

    in_specs = [
        pl.BlockSpec((BM, K), lambda m: (m, 0)),
        pl.BlockSpec((HID, K), lambda m: (0, 0)),
        pl.BlockSpec((1, HID), lambda m: (0, 0)),
        pl.BlockSpec((NOUT, HID), lambda m: (0, 0)),
        pl.BlockSpec((1, NOUT), lambda m: (0, 0)),
    ]
    args = [flat, W1b, b1r, W2b, b2r]
    kwargs = {}
    if out_prev is not None:
        in_specs.append(pl.BlockSpec(memory_space=pl.ANY))
        args.append(out_prev)
        kwargs["input_output_aliases"] = {5: 0}

    return pl.pallas_call(
        body,
        grid=(Mc,),
        in_specs=in_specs,
        out_specs=pl.BlockSpec((BM, NOUT), lambda m: (m + block_off, 0)),
        out_shape=jax.ShapeDtypeStruct((B, NOUT), jnp.float32),
        **kwargs,
    )(*args)


def kernel(sentence, emb_table, W1, b1, W2, b2):
    B, S = sentence.shape
    V, D = emb_table.shape
    HID = W1.shape[0]
    T = W2.shape[0]

    W1b = _cast_bf16(W1, 512)
    W2b = _cast_bf16(W2, T)
    b1r = b1.reshape(1, HID)
    b2r = b2.reshape(1, T)

    perm = (
        sentence.astype(jnp.int32)
        .reshape(B // 8, 8, S)
        .transpose(0, 2, 1)
        .reshape(B * S)
    )

    Bc = B // 2
    CH = 80
    NCH = (Bc * S) // (_NW * CH)
    half = Bc * S
    flat0 = _sc_gather_tiled(
        emb_table, perm[:half].reshape(_NW, NCH, CH), Bc, S)
    flat1 = _sc_gather_tiled(
        emb_table, perm[half:].reshape(_NW, NCH, CH), Bc, S)
    out0 = _tc_mlp(flat0, W1b, b1r, W2b, b2r, B, 0)
    return _tc_mlp(flat1, W1b, b1r, W2b, b2r, B, Bc // 256, out_prev=out0)


# speedup vs baseline: 1.0453x; 1.0453x over previous
import functools

import jax
import jax.numpy as jnp
from jax import lax
from jax.experimental import pallas as pl
from jax.experimental.pallas import tpu as pltpu
from jax.experimental.pallas import tpu_sc as plsc

_NC = 2
_NS = 16
_NW = _NC * _NS


def _cast_bf16(W, BH):
    H, K = W.shape

    def body(w_ref, o_ref):
        o_ref[...] = w_ref[...].astype(jnp.bfloat16)

    return pl.pallas_call(
        body,
        grid=(H // BH,),
        in_specs=[pl.BlockSpec((BH, K), lambda h: (h, 0))],
        out_specs=pl.BlockSpec((BH, K), lambda h: (h, 0)),
        out_shape=jax.ShapeDtypeStruct((H, K), jnp.bfloat16),
    )(W)


def _sc_gather_tiled(table, idx3, B, S):
    V, D = table.shape
    NW, NCH, CH = idx3.shape
    assert NCH % 2 == 0 and CH % 8 == 0
    b_per_w = NCH * CH
    NT = CH // 8

    mesh = plsc.VectorSubcoreMesh(core_axis_name="c", subcore_axis_name="s")
    NSLOT = 4
    assert NCH % NSLOT == 0

    @functools.partial(
        pl.kernel,
        mesh=mesh,
        compiler_params=pltpu.CompilerParams(use_tc_tiling_on_sc=True),
        out_type=jax.ShapeDtypeStruct((B, S * D), table.dtype),
        scratch_types=[
            pltpu.VMEM((NCH, CH), jnp.int32),
            pltpu.VMEM((4, CH, D), table.dtype),
        ] + [pltpu.SemaphoreType.DMA] * 8,
    )
    def k(table_hbm, idx_hbm, out_hbm, idx_v, rows_v, *sems):
        gsems = sems[:NSLOT]
        osems = sems[NSLOT:]
        c = lax.axis_index("c")
        s = lax.axis_index("s")
        wid = s * _NC + c
        base = wid * b_per_w
        pltpu.sync_copy(idx_hbm.at[wid], idx_v)
        pltpu.async_copy(table_hbm.at[idx_v.at[0]], rows_v.at[0], gsems[0])
        pltpu.async_copy(table_hbm.at[idx_v.at[1]], rows_v.at[1], gsems[1])

        def body(g, carry):
            for b in range(NSLOT):
                j = NSLOT * g + b
                pltpu.make_async_copy(
                    table_hbm.at[pl.ds(0, CH)], rows_v.at[b], gsems[b]
                ).wait()
                t0 = (base + j * CH) // 8
                for kk in range(NT):
                    t = t0 + kk
                    bt = t // S
                    cc = t % S
                    pltpu.async_copy(
                        rows_v.at[b, pl.ds(kk * 8, 8)],
                        out_hbm.at[pl.ds(bt * 8, 8), pl.ds(cc * D, D)],
                        osems[b],
                    )
                jn = j + 2
                bn = (b + 2) % NSLOT

                @pl.when(jnp.logical_and(jn < NCH, jn >= NSLOT))
                def _():
                    pltpu.make_async_copy(
                        table_hbm.at[pl.ds(0, CH)], rows_v.at[bn], osems[bn]
                    ).wait()

                @pl.when(jn < NCH)
                def _():
                    pltpu.async_copy(
                        table_hbm.at[idx_v.at[jn]], rows_v.at[bn], gsems[bn]
                    )

            return carry

        lax.fori_loop(0, NCH // NSLOT, body, 0)

        for b in range(NSLOT):
            pltpu.make_async_copy(
                table_hbm.at[pl.ds(0, CH)], rows_v.at[b], osems[b]
            ).wait()

    return k(table, idx3)


def _tc_mlp(flat, W1b, b1r, W2b, b2r, B, block_off, out_prev=None):
    Bc, K = flat.shape
    HID = W1b.shape[0]
    NOUT = W2b.shape[0]
    BM = 256
    Mc = Bc // BM

    def body(*refs):
        x_ref, w1_ref, b1_ref, w2_ref, b2_ref = refs[:5]
        o_ref = refs[-1]
        x = x_ref[...].astype(jnp.bfloat16)
        h = lax.dot_general(
            x, w1_ref[...], (((1,), (1,)), ((), ())),
            preferred_element_type=jnp.float32,
        )
        h = jnp.maximum(h + b1_ref[...], 0.0).astype(jnp.bfloat16)
        o = lax.dot_general(
            h, w2_ref[...].astype(jnp.bfloat16), (((1,), (1,)), ((), ())),
            preferred_element_type=jnp.float32,
        )
        o_ref[...] = o + b2_ref[...]

    in_specs = [
        pl.BlockSpec((BM, K), lambda m: (m, 0)),
        pl.BlockSpec((HID, K), lambda m: (0, 0)),
        pl.BlockSpec((1, HID), lambda m: (0, 0)),
        pl.BlockSpec((NOUT, HID), lambda m: (0, 0)),
        pl.BlockSpec((1, NOUT), lambda m: (0, 0)),
    ]
    args = [flat, W1b, b1r, W2b, b2r]
    kwargs = {}
    if out_prev is not None:
        in_specs.append(pl.BlockSpec(memory_space=pl.ANY))
        args.append(out_prev)
        kwargs["input_output_aliases"] = {5: 0}

    return pl.pallas_call(
        body,
        grid=(Mc,),
        in_specs=in_specs,
        out_specs=pl.BlockSpec((BM, NOUT), lambda m: (m + block_off, 0)),
        out_shape=jax.ShapeDtypeStruct((B, NOUT), jnp.float32),
        **kwargs,
    )(*args)


def kernel(sentence, emb_table, W1, b1, W2, b2):
    B, S = sentence.shape
    V, D = emb_table.shape
    HID = W1.shape[0]
    T = W2.shape[0]

    W1b = _cast_bf16(W1, 512)
    b1r = b1.reshape(1, HID)
    b2r = b2.reshape(1, T)

    Bc = B // 2
    CH = 80
    NCH = (Bc * S) // (_NW * CH)

    def chunk_idx(c):
        return (
            sentence[c * Bc:(c + 1) * Bc].astype(jnp.int32)
            .reshape(Bc // 8, 8, S)
            .transpose(0, 2, 1)
            .reshape(_NW, NCH, CH)
        )

    flat0 = _sc_gather_tiled(emb_table, chunk_idx(0), Bc, S)
    flat1 = _sc_gather_tiled(emb_table, chunk_idx(1), Bc, S)
    out0 = _tc_mlp(flat0, W1b, b1r, W2, b2r, B, 0)
    return _tc_mlp(flat1, W1b, b1r, W2, b2r, B, Bc // 256, out_prev=out0)
